# baseline (speedup 1.0000x reference)
import functools
import math

import jax
import jax.numpy as jnp
from jax import lax
from jax.experimental import pallas as pl
from jax.experimental.pallas import tpu as pltpu
from jax.experimental.pallas import tpu_sc as plsc

D_MODEL = 64
B, S = 4096, 200

NC, NS = 2, 16
NW = NC * NS
B_PER_W = B // NW

NB = 4
CH = NB * S
G = B_PER_W // NB
NBUF = 2
S0 = 128
S1 = S - S0

_mesh = plsc.VectorSubcoreMesh(core_axis_name="c", subcore_axis_name="s")


@functools.partial(
    pl.kernel,
    mesh=_mesh,
    out_type=jax.ShapeDtypeStruct((B, S, D_MODEL), jnp.float32),
    scratch_types=[
        pltpu.VMEM((NBUF, NB, S), jnp.int32),
        pltpu.VMEM((NBUF, NB, S, D_MODEL), jnp.float32),
        [pltpu.SemaphoreType.DMA] * NBUF,
        [pltpu.SemaphoreType.DMA] * NBUF,
    ],
    compiler_params=pltpu.CompilerParams(use_tc_tiling_on_sc=False),
)
def _emb_lookup(x_hbm, table_hbm, out_hbm, idx_v, rows_v, gsems, wsems):
    wid = lax.axis_index("s") * NC + lax.axis_index("c")
    b_base = wid * B_PER_W

    def start_gathers(h, b):
        pltpu.sync_copy(x_hbm.at[pl.ds(b_base + h * NB, NB)], idx_v.at[b])
        for r in range(NB):
            pltpu.async_copy(
                table_hbm.at[idx_v.at[b, r, pl.ds(0, S0)]],
                rows_v.at[b, r].at[pl.ds(0, S0)],
                gsems[b],
            )
            pltpu.async_copy(
                table_hbm.at[idx_v.at[b, r, pl.ds(S0, S1)]],
                rows_v.at[b, r].at[pl.ds(S0, S1)],
                gsems[b],
            )

    def drain_gathers(b):
        pltpu.make_async_copy(
            out_hbm.at[pl.ds(0, NB)], rows_v.at[b], gsems[b]
        ).wait()

    def drain_writeback(b):
        pltpu.make_async_copy(
            rows_v.at[b], out_hbm.at[pl.ds(0, NB)], wsems[b]
        ).wait()

    def scale(b):
        for rb in range(NB):
            def body(r, c2, rb=rb):
                for q in range(D_MODEL // 16):
                    sl = pl.ds(q * 16, 16)
                    rows_v[b, rb, r, sl] = rows_v[b, rb, r, sl] * 8.0
                return c2

            lax.fori_loop(0, S, body, 0, unroll=4)

    def process(h, b):
        @pl.when(h + 1 < G)
        def _prefetch():
            @pl.when(h >= 1)
            def _():
                drain_writeback(1 - b)

            start_gathers(h + 1, 1 - b)

        drain_gathers(b)
        scale(b)
        pltpu.async_copy(
            rows_v.at[b],
            out_hbm.at[pl.ds(b_base + h * NB, NB)],
            wsems[b],
        )

    start_gathers(0, 0)

    def outer(g, carry):
        for b in range(NBUF):
            process(g + b, b)
        return carry

    lax.fori_loop(0, G // NBUF, lambda i, c: outer(i * NBUF, c), 0)
    drain_writeback(NBUF - 2)
    drain_writeback(NBUF - 1)


def kernel(x, table):
    return _emb_lookup(x.astype(jnp.int32), table)
